# baseline (speedup 1.0000x reference)
import functools

import jax
import jax.numpy as jnp
from jax import lax
from jax.experimental import pallas as pl
from jax.experimental.pallas import tpu as pltpu
from jax.experimental.pallas import tpu_sc as plsc

B = 16384
D = 64
DP = 128
L = 16
NW = 32
BW = B // NW
CHUNK = 128
NCH = BW // CHUNK
NPAIR = 50000


@functools.partial(
    pl.kernel,
    out_type=[
        jax.ShapeDtypeStruct((B,), jnp.float32),
        jax.ShapeDtypeStruct((B,), jnp.float32),
        jax.ShapeDtypeStruct((B,), jnp.float32),
    ],
    mesh=plsc.VectorSubcoreMesh(core_axis_name="c", subcore_axis_name="s"),
    compiler_params=pltpu.CompilerParams(
        needs_layout_passes=False, use_tc_tiling_on_sc=False),
    scratch_types=[
        pltpu.VMEM((NCH, CHUNK), jnp.int32),
        pltpu.VMEM((NCH, CHUNK), jnp.int32),
        pltpu.VMEM((NCH, CHUNK), jnp.int32),
        pltpu.VMEM((NCH, CHUNK), jnp.int32),
        pltpu.VMEM((BW + L,), jnp.int32),
        pltpu.VMEM((BW + L,), jnp.int32),
        pltpu.VMEM((BW + L,), jnp.int32),
        pltpu.VMEM((2, CHUNK, DP), jnp.bfloat16),
        pltpu.VMEM((2, CHUNK, DP), jnp.bfloat16),
        pltpu.VMEM((2, CHUNK, DP), jnp.bfloat16),
        pltpu.VMEM((BW,), jnp.float32),
        pltpu.VMEM((BW,), jnp.float32),
        pltpu.VMEM((BW,), jnp.float32),
        pltpu.SemaphoreType.DMA,
        pltpu.SemaphoreType.DMA,
    ],
)
def _ubpr_sc(pu_hbm, pi_hbm, pj_hbm, hu_hbm, hi_hbm, hj_hbm, bi_hbm,
             ue_hbm, ie_hbm, prop_hbm,
             pos_hbm, neg_hbm, ppos_hbm,
             idx_u, idx_i, idx_j, idx_b, h_u, h_i, h_j, u_v, i_v, j_v,
             prop_v, pos_v, neg_v, sem0, sem1):
    wid = lax.axis_index("s") * 2 + lax.axis_index("c")
    base = wid * BW
    sems = (sem0, sem1)

    for c in range(NCH):
        src = pl.ds(base + c * CHUNK, CHUNK)
        pltpu.sync_copy(pu_hbm.at[src], idx_u.at[c])
        pltpu.sync_copy(pi_hbm.at[src], idx_i.at[c])
        pltpu.sync_copy(pj_hbm.at[src], idx_j.at[c])
        pltpu.sync_copy(bi_hbm.at[src], idx_b.at[c])
    full = pl.ds(base, BW)
    dst = pl.ds(0, BW)
    pltpu.sync_copy(hu_hbm.at[full], h_u.at[dst])
    pltpu.sync_copy(hi_hbm.at[full], h_i.at[dst])
    pltpu.sync_copy(hj_hbm.at[full], h_j.at[dst])

    def fire(c):
        buf = c % 2
        sem = sems[buf]
        return [
            pltpu.async_copy(ue_hbm.at[idx_u.at[c]], u_v.at[buf], sem),
            pltpu.async_copy(ie_hbm.at[idx_i.at[c]], i_v.at[buf], sem),
            pltpu.async_copy(ie_hbm.at[idx_j.at[c]], j_v.at[buf], sem),
            pltpu.async_copy(prop_hbm.at[idx_b.at[c]],
                             prop_v.at[pl.ds(c * CHUNK, CHUNK)], sem),
        ]

    lane = lax.iota(jnp.int32, L)
    last = lane == (L - 1)

    pending = fire(0)
    for c in range(NCH):
        for cp in pending:
            cp.wait()
        if c + 1 < NCH:
            pending = fire(c + 1)
        buf = c % 2
        ubuf = u_v.at[buf]
        ibuf = i_v.at[buf]
        jbuf = j_v.at[buf]
        out_off = c * CHUNK

        @plsc.parallel_loop(0, CHUNK, 1, unroll=8)
        def _elem(e):
            ge = out_off + e
            hu = h_u[pl.ds(ge, L)][0]
            hi = h_i[pl.ds(ge, L)][0]
            hj = h_j[pl.ds(ge, L)][0]
            acc_p = None
            acc_n = None
            for k in range(2):
                u0, u1 = plsc.unpack(ubuf[e, pl.ds(hu + k * 32, 32)],
                                     format=plsc.PackFormat.INTERLEAVED)
                i0, i1 = plsc.unpack(ibuf[e, pl.ds(hi + k * 32, 32)],
                                     format=plsc.PackFormat.INTERLEAVED)
                j0, j1 = plsc.unpack(jbuf[e, pl.ds(hj + k * 32, 32)],
                                     format=plsc.PackFormat.INTERLEAVED)
                tp = u0 * i0 + u1 * i1
                tn = u0 * j0 + u1 * j1
                acc_p = tp if acc_p is None else acc_p + tp
                acc_n = tn if acc_n is None else acc_n + tn
            eidx = jnp.full((L,), ge, jnp.int32)
            plsc.store_scatter(pos_v, [eidx], plsc.cumsum(acc_p), mask=last)
            plsc.store_scatter(neg_v, [eidx], plsc.cumsum(acc_n), mask=last)

    def clamp(g, _):
        sl = pl.ds(g * L, L)
        prop_v[sl] = jnp.maximum(prop_v[sl], 0.1)
        return 0

    lax.fori_loop(0, BW // L, clamp, 0, unroll=False)

    out = pl.ds(base, BW)
    pltpu.sync_copy(pos_v, pos_hbm.at[out])
    pltpu.sync_copy(neg_v, neg_hbm.at[out])
    pltpu.sync_copy(prop_v, ppos_hbm.at[out])


@jax.jit
def kernel(batch_user, batch_pos_item, batch_neg_item, user_emb, item_emb,
           i_propensity):
    bu = batch_user.astype(jnp.int32)
    bi = batch_pos_item.astype(jnp.int32)
    bj = batch_neg_item.astype(jnp.int32)
    uep = user_emb.astype(jnp.bfloat16).reshape(NPAIR, DP)
    iep = item_emb.astype(jnp.bfloat16).reshape(NPAIR, DP)
    pos, neg, ppos = _ubpr_sc(
        bu >> 1, bi >> 1, bj >> 1,
        (bu & 1) << 6, (bi & 1) << 6, (bj & 1) << 6,
        bi, uep, iep, i_propensity)
    return pos.reshape(B, 1), neg.reshape(B, 1), ppos


# speedup vs baseline: 1.4208x; 1.4208x over previous
import functools

import jax
import jax.numpy as jnp
from jax import lax
from jax.experimental import pallas as pl
from jax.experimental.pallas import tpu as pltpu
from jax.experimental.pallas import tpu_sc as plsc

B = 16384
D = 64
DP = 64
L = 16
NW = 32
BW = B // NW
CHUNK = 128
NCH = BW // CHUNK


@functools.partial(
    pl.kernel,
    out_type=[
        jax.ShapeDtypeStruct((B,), jnp.float32),
        jax.ShapeDtypeStruct((B,), jnp.float32),
        jax.ShapeDtypeStruct((B,), jnp.float32),
    ],
    mesh=plsc.VectorSubcoreMesh(core_axis_name="c", subcore_axis_name="s"),
    compiler_params=pltpu.CompilerParams(
        needs_layout_passes=False, use_tc_tiling_on_sc=False),
    scratch_types=[
        pltpu.VMEM((NCH, CHUNK), jnp.int32),
        pltpu.VMEM((NCH, CHUNK), jnp.int32),
        pltpu.VMEM((NCH, CHUNK), jnp.int32),
        pltpu.VMEM((2, CHUNK, DP), jnp.float32),
        pltpu.VMEM((2, CHUNK, DP), jnp.float32),
        pltpu.VMEM((2, CHUNK, DP), jnp.float32),
        pltpu.VMEM((BW,), jnp.float32),
        pltpu.VMEM((BW,), jnp.float32),
        pltpu.VMEM((BW,), jnp.float32),
        pltpu.SemaphoreType.DMA,
        pltpu.SemaphoreType.DMA,
        pltpu.SemaphoreType.DMA,
    ],
)
def _ubpr_sc(bu_hbm, bi_hbm, bj_hbm, ue_hbm, ie_hbm, prop_hbm,
             pos_hbm, neg_hbm, ppos_hbm,
             idx_u, idx_i, idx_j, u_v, i_v, j_v, prop_v, pos_v, neg_v,
             sem0, sem1, sem_idx):
    wid = lax.axis_index("s") * 2 + lax.axis_index("c")
    base = wid * BW
    sems = (sem0, sem1)

    idx_copies = []
    for c in range(NCH):
        src = pl.ds(base + c * CHUNK, CHUNK)
        idx_copies.append(pltpu.async_copy(bu_hbm.at[src], idx_u.at[c], sem_idx))
        idx_copies.append(pltpu.async_copy(bi_hbm.at[src], idx_i.at[c], sem_idx))
        idx_copies.append(pltpu.async_copy(bj_hbm.at[src], idx_j.at[c], sem_idx))
    for cp in idx_copies:
        cp.wait()

    def fire(c):
        buf = c % 2
        sem = sems[buf]
        return [
            pltpu.async_copy(ue_hbm.at[idx_u.at[c]], u_v.at[buf], sem),
            pltpu.async_copy(ie_hbm.at[idx_i.at[c]], i_v.at[buf], sem),
            pltpu.async_copy(ie_hbm.at[idx_j.at[c]], j_v.at[buf], sem),
            pltpu.async_copy(prop_hbm.at[idx_i.at[c]],
                             prop_v.at[pl.ds(c * CHUNK, CHUNK)], sem),
        ]

    lane = lax.iota(jnp.int32, L)
    last = lane == (L - 1)

    pending = fire(0)
    for c in range(NCH):
        for cp in pending:
            cp.wait()
        if c + 1 < NCH:
            pending = fire(c + 1)
        buf = c % 2
        ub = u_v.at[buf]
        ib = i_v.at[buf]
        jb = j_v.at[buf]
        out_off = c * CHUNK

        @plsc.parallel_loop(0, CHUNK, 1, unroll=16)
        def _elem(e):
            acc_p = None
            acc_n = None
            for k in range(D // L):
                sl = pl.ds(k * L, L)
                uu = ub[e, sl]
                pp = uu * ib[e, sl]
                nn = uu * jb[e, sl]
                acc_p = pp if acc_p is None else acc_p + pp
                acc_n = nn if acc_n is None else acc_n + nn
            eidx = jnp.full((L,), out_off + e, jnp.int32)
            plsc.store_scatter(pos_v, [eidx], plsc.cumsum(acc_p), mask=last)
            plsc.store_scatter(neg_v, [eidx], plsc.cumsum(acc_n), mask=last)

    def clamp(g, _):
        sl = pl.ds(g * L, L)
        prop_v[sl] = jnp.maximum(prop_v[sl], 0.1)
        return 0

    lax.fori_loop(0, BW // L, clamp, 0, unroll=False)

    out = pl.ds(base, BW)
    pltpu.sync_copy(pos_v, pos_hbm.at[out])
    pltpu.sync_copy(neg_v, neg_hbm.at[out])
    pltpu.sync_copy(prop_v, ppos_hbm.at[out])


@jax.jit
def kernel(batch_user, batch_pos_item, batch_neg_item, user_emb, item_emb,
           i_propensity):
    bu = batch_user.astype(jnp.int32)
    bi = batch_pos_item.astype(jnp.int32)
    bj = batch_neg_item.astype(jnp.int32)
    pos, neg, ppos = _ubpr_sc(bu, bi, bj, user_emb, item_emb, i_propensity)
    return pos.reshape(B, 1), neg.reshape(B, 1), ppos
